# baseline (speedup 1.0000x reference)
import functools

import jax
import jax.numpy as jnp
from jax import lax
from jax.experimental import pallas as pl

N = 5000
NMS_THRESH = 0.3
TOPK = 300
B = 512
NB = 10
NP = B * NB
OUT_R = 304
F32 = jnp.float32


def _nms_body(data_ref, dataT_ref, out_ref):
    data = data_ref[:, :]
    dataT = dataT_ref[:, :]

    k_col = lax.bitcast_convert_type(data[:, 4:5], jnp.int32)
    k_row = lax.bitcast_convert_type(dataT[4:5, :], jnp.int32)
    idx_col = lax.broadcasted_iota(jnp.int32, (NP, 1), 0)
    idx_row = lax.broadcasted_iota(jnp.int32, (1, NP), 1)

    rank_row_parts = []
    for t in range(NB):
        sl = slice(t * B, (t + 1) * B)
        kb_row = k_row[:, sl]
        cnt = jnp.zeros((1, B), jnp.int32)
        if t > 0:
            d_above = k_col[:t * B, :] - kb_row
            cnt = cnt + jnp.sum((d_above >= 0).astype(jnp.int32),
                                axis=0, keepdims=True)
        dd = k_col[sl, :] - kb_row
        diag = (dd > 0) | ((dd == 0) & (idx_col[sl, :] < idx_row[:, sl]))
        cnt = cnt + jnp.sum(diag.astype(jnp.int32), axis=0, keepdims=True)
        if t < NB - 1:
            d_below = k_col[(t + 1) * B:, :] - kb_row
            cnt = cnt + jnp.sum((d_below > 0).astype(jnp.int32),
                                axis=0, keepdims=True)
        rank_row_parts.append(cnt)
    rank_row = jnp.concatenate(rank_row_parts, axis=1)

    bh = data.astype(jnp.bfloat16)
    r1 = data - bh.astype(F32)
    bm = r1.astype(jnp.bfloat16)
    bl = (r1 - bm.astype(F32)).astype(jnp.bfloat16)
    data3 = jnp.concatenate([bh, bm, bl], axis=1)
    sorted_parts = []
    for t in range(NB):
        sl = slice(t * B, (t + 1) * B)
        oh = (rank_row == idx_col[sl, :]).astype(jnp.bfloat16)
        p3 = jnp.dot(oh, data3, preferred_element_type=F32)
        sorted_parts.append(p3[:, 0:8] + p3[:, 8:16] + p3[:, 16:24])
    sdata = jnp.concatenate(sorted_parts, axis=0)
    sdataT = jnp.transpose(sdata)

    x1r = sdataT[0:1, :]
    y1r = sdataT[1:2, :]
    x2r = sdataT[2:3, :]
    y2r = sdataT[3:4, :]
    area_row = (x2r - x1r) * (y2r - y1r)

    BN = 1024
    NBN = NP // BN
    bi_col = lax.broadcasted_iota(jnp.int32, (BN, 1), 0)
    bj_row = lax.broadcasted_iota(jnp.int32, (1, BN), 1)
    tri_strict = (bi_col < bj_row)

    keep_blocks = [jnp.ones((1, BN), F32) for _ in range(NBN)]
    for t in range(NBN):
        sl = slice(t * BN, (t + 1) * BN)
        x1c = sdata[sl, 0:1]
        y1c = sdata[sl, 1:2]
        x2c = sdata[sl, 2:3]
        y2c = sdata[sl, 3:4]
        area_col = (x2c - x1c) * (y2c - y1c)

        def _iou_vs(slc):
            ix1 = jnp.maximum(x1c, x1r[:, slc])
            iy1 = jnp.maximum(y1c, y1r[:, slc])
            ix2 = jnp.minimum(x2c, x2r[:, slc])
            iy2 = jnp.minimum(y2c, y2r[:, slc])
            iw = jnp.maximum(ix2 - ix1, 0.0)
            ih = jnp.maximum(iy2 - iy1, 0.0)
            inter = iw * ih
            union = area_col + area_row[:, slc] - inter
            return inter / jnp.maximum(union, 1e-8)

        cf = ((_iou_vs(sl) > NMS_THRESH) & tri_strict).astype(F32)
        keep_in = keep_blocks[t]

        def _step(k):
            supcnt = jnp.dot(k, cf, preferred_element_type=F32)
            return keep_in * (supcnt == 0.0).astype(F32)

        def _cond(st):
            return st[1]

        def _body(st):
            k1 = _step(st[0])
            k2 = _step(k1)
            return (k2, jnp.any(k2 != k1))

        keep_blk, _ = lax.while_loop(_cond, _body,
                                     (keep_in, jnp.bool_(True)))
        keep_blocks[t] = keep_blk

        if t < NBN - 1:
            sl_rest = slice((t + 1) * BN, NP)
            mf = (_iou_vs(sl_rest) > NMS_THRESH).astype(F32)
            supcnt = jnp.dot(keep_blk, mf, preferred_element_type=F32)
            alive = (supcnt == 0.0).astype(F32)
            for u in range(t + 1, NBN):
                lo = (u - t - 1) * BN
                keep_blocks[u] = keep_blocks[u] * alive[:, lo:lo + BN]

    keep = jnp.concatenate(keep_blocks, axis=1)

    pos_row = idx_row.astype(F32)
    valid = (pos_row < float(N)).astype(F32)
    kv = keep * valid
    tri_b = (lax.broadcasted_iota(jnp.int32, (B, 1), 0) <
             lax.broadcasted_iota(jnp.int32, (1, B), 1)).astype(F32)
    prefk_parts = []
    offset = jnp.zeros((1, 1), F32)
    for t in range(NB):
        sl = slice(t * B, (t + 1) * B)
        kvb = kv[:, sl]
        within = jnp.dot(kvb, tri_b, preferred_element_type=F32)
        prefk_parts.append(within + offset)
        offset = offset + jnp.sum(kvb, keepdims=True)
    prefk = jnp.concatenate(prefk_parts, axis=1)
    ktot = offset
    prefs = jnp.minimum(pos_row, float(N)) - prefk
    dest = jnp.where(kv > 0.0, prefk, ktot + prefs)
    dest = jnp.where(valid > 0.0, dest, 2.0 * NP)

    sh = sdata.astype(jnp.bfloat16)
    t1 = sdata - sh.astype(F32)
    sm = t1.astype(jnp.bfloat16)
    sl3 = (t1 - sm.astype(F32)).astype(jnp.bfloat16)
    sdata3 = jnp.concatenate([sh, sm, sl3], axis=1)
    r_col = lax.broadcasted_iota(jnp.int32, (OUT_R, 1), 0).astype(F32)
    oh_out = (dest == r_col).astype(jnp.bfloat16)
    q3 = jnp.dot(oh_out, sdata3, preferred_element_type=F32)
    out_ref[:, :] = q3[:, 0:8] + q3[:, 8:16] + q3[:, 16:24]


def _nms_call(data, dataT, interpret=False):
    return pl.pallas_call(
        _nms_body,
        out_shape=jax.ShapeDtypeStruct((OUT_R, 8), F32),
        interpret=interpret,
    )(data, dataT)


@jax.jit
def kernel(boxes, scores):
    boxes_p = jnp.concatenate(
        [boxes.astype(F32), jnp.zeros((NP - N, 4), F32)], axis=0)
    scores_p = jnp.concatenate(
        [scores.astype(F32), jnp.zeros((NP - N,), F32)], axis=0)
    data = jnp.concatenate(
        [boxes_p, scores_p[:, None], jnp.zeros((NP, 3), F32)], axis=1)
    out = _nms_call(data, data.T)
    return out[:TOPK, :5]


# speedup vs baseline: 1.0361x; 1.0361x over previous
import functools

import jax
import jax.numpy as jnp
from jax import lax
from jax.experimental import pallas as pl

N = 5000
NMS_THRESH = 0.3
TOPK = 300
B = 512
NB = 10
NP = B * NB
OUT_R = 304
F32 = jnp.float32


def _nms_body(data_ref, dataT_ref, out_ref):
    data = data_ref[:, :]
    dataT = dataT_ref[:, :]

    k_col = lax.bitcast_convert_type(data[:, 4:5], jnp.int32)
    k_row = lax.bitcast_convert_type(dataT[4:5, :], jnp.int32)
    idx_col = lax.broadcasted_iota(jnp.int32, (NP, 1), 0)
    idx_row = lax.broadcasted_iota(jnp.int32, (1, NP), 1)

    rank_row_parts = []
    for t in range(NB):
        sl = slice(t * B, (t + 1) * B)
        kb_row = k_row[:, sl]
        cnt = jnp.zeros((1, B), jnp.int32)
        if t > 0:
            d_above = k_col[:t * B, :] - kb_row
            cnt = cnt + jnp.sum((d_above >= 0).astype(jnp.int32),
                                axis=0, keepdims=True)
        dd = k_col[sl, :] - kb_row
        diag = (dd > 0) | ((dd == 0) & (idx_col[sl, :] < idx_row[:, sl]))
        cnt = cnt + jnp.sum(diag.astype(jnp.int32), axis=0, keepdims=True)
        if t < NB - 1:
            d_below = k_col[(t + 1) * B:, :] - kb_row
            cnt = cnt + jnp.sum((d_below > 0).astype(jnp.int32),
                                axis=0, keepdims=True)
        rank_row_parts.append(cnt)
    rank_row = jnp.concatenate(rank_row_parts, axis=1)

    bh = data.astype(jnp.bfloat16)
    r1 = data - bh.astype(F32)
    bm = r1.astype(jnp.bfloat16)
    bl = (r1 - bm.astype(F32)).astype(jnp.bfloat16)
    data3 = jnp.concatenate([bh, bm, bl], axis=1)
    sorted_parts = []
    for t in range(NB):
        sl = slice(t * B, (t + 1) * B)
        oh = (rank_row == idx_col[sl, :]).astype(jnp.bfloat16)
        p3 = jnp.dot(oh, data3, preferred_element_type=F32)
        sorted_parts.append(p3[:, 0:8] + p3[:, 8:16] + p3[:, 16:24])
    sdata = jnp.concatenate(sorted_parts, axis=0)
    sdataT = jnp.transpose(sdata)

    x1r = sdataT[0:1, :]
    y1r = sdataT[1:2, :]
    x2r = sdataT[2:3, :]
    y2r = sdataT[3:4, :]
    area_row = (x2r - x1r) * (y2r - y1r)

    BN = 512
    NBN = NP // BN
    bi_col = lax.broadcasted_iota(jnp.int32, (BN, 1), 0)
    bj_row = lax.broadcasted_iota(jnp.int32, (1, BN), 1)
    tri_strict = (bi_col < bj_row)

    keep_blocks = [jnp.ones((1, BN), F32) for _ in range(NBN)]
    for t in range(NBN):
        sl = slice(t * BN, (t + 1) * BN)
        x1c = sdata[sl, 0:1]
        y1c = sdata[sl, 1:2]
        x2c = sdata[sl, 2:3]
        y2c = sdata[sl, 3:4]
        area_col = (x2c - x1c) * (y2c - y1c)

        def _iou_vs(slc):
            ix1 = jnp.maximum(x1c, x1r[:, slc])
            iy1 = jnp.maximum(y1c, y1r[:, slc])
            ix2 = jnp.minimum(x2c, x2r[:, slc])
            iy2 = jnp.minimum(y2c, y2r[:, slc])
            iw = jnp.maximum(ix2 - ix1, 0.0)
            ih = jnp.maximum(iy2 - iy1, 0.0)
            inter = iw * ih
            union = area_col + area_row[:, slc] - inter
            return inter / jnp.maximum(union, 1e-8)

        cf = ((_iou_vs(sl) > NMS_THRESH) & tri_strict).astype(F32)
        keep_in = keep_blocks[t]

        def _step(k):
            supcnt = jnp.dot(k, cf, preferred_element_type=F32)
            return keep_in * (supcnt == 0.0).astype(F32)

        def _cond(st):
            return st[1]

        def _body(st):
            k1 = _step(st[0])
            k2 = _step(k1)
            return (k2, jnp.any(k2 != k1))

        keep_blk, _ = lax.while_loop(_cond, _body,
                                     (keep_in, jnp.bool_(True)))
        keep_blocks[t] = keep_blk

        if t < NBN - 1:
            sl_rest = slice((t + 1) * BN, NP)
            mf = (_iou_vs(sl_rest) > NMS_THRESH).astype(F32)
            supcnt = jnp.dot(keep_blk, mf, preferred_element_type=F32)
            alive = (supcnt == 0.0).astype(F32)
            for u in range(t + 1, NBN):
                lo = (u - t - 1) * BN
                keep_blocks[u] = keep_blocks[u] * alive[:, lo:lo + BN]

    keep = jnp.concatenate(keep_blocks, axis=1)

    pos_row = idx_row.astype(F32)
    valid = (pos_row < float(N)).astype(F32)
    kv = keep * valid
    tri_b = (lax.broadcasted_iota(jnp.int32, (B, 1), 0) <
             lax.broadcasted_iota(jnp.int32, (1, B), 1)).astype(F32)
    prefk_parts = []
    offset = jnp.zeros((1, 1), F32)
    for t in range(NB):
        sl = slice(t * B, (t + 1) * B)
        kvb = kv[:, sl]
        within = jnp.dot(kvb, tri_b, preferred_element_type=F32)
        prefk_parts.append(within + offset)
        offset = offset + jnp.sum(kvb, keepdims=True)
    prefk = jnp.concatenate(prefk_parts, axis=1)
    ktot = offset
    prefs = jnp.minimum(pos_row, float(N)) - prefk
    dest = jnp.where(kv > 0.0, prefk, ktot + prefs)
    dest = jnp.where(valid > 0.0, dest, 2.0 * NP)

    sh = sdata.astype(jnp.bfloat16)
    t1 = sdata - sh.astype(F32)
    sm = t1.astype(jnp.bfloat16)
    sl3 = (t1 - sm.astype(F32)).astype(jnp.bfloat16)
    sdata3 = jnp.concatenate([sh, sm, sl3], axis=1)
    r_col = lax.broadcasted_iota(jnp.int32, (OUT_R, 1), 0).astype(F32)
    oh_out = (dest == r_col).astype(jnp.bfloat16)
    q3 = jnp.dot(oh_out, sdata3, preferred_element_type=F32)
    out_ref[:, :] = q3[:, 0:8] + q3[:, 8:16] + q3[:, 16:24]


def _nms_call(data, dataT, interpret=False):
    return pl.pallas_call(
        _nms_body,
        out_shape=jax.ShapeDtypeStruct((OUT_R, 8), F32),
        interpret=interpret,
    )(data, dataT)


@jax.jit
def kernel(boxes, scores):
    boxes_p = jnp.concatenate(
        [boxes.astype(F32), jnp.zeros((NP - N, 4), F32)], axis=0)
    scores_p = jnp.concatenate(
        [scores.astype(F32), jnp.zeros((NP - N,), F32)], axis=0)
    data = jnp.concatenate(
        [boxes_p, scores_p[:, None], jnp.zeros((NP, 3), F32)], axis=1)
    out = _nms_call(data, data.T)
    return out[:TOPK, :5]
